# Initial kernel scaffold; baseline (speedup 1.0000x reference)
import jax
import jax.numpy as jnp
from jax.experimental import pallas as pl


def kernel(input, W1, b1, W2, b2, Wfc1, bfc1, Wtfc1, btfc1, spatial_edge_index, temporal_edge_index):
    raise NotImplementedError("write your pallas kernel here")


# speedup vs baseline: 1.3427x; 1.3427x over previous
import jax
import jax.numpy as jnp
from jax.experimental import pallas as pl
from jax.experimental.pallas import tpu as pltpu

THRESH = 0.5
DECAY = 0.2
K = 3

B = 64
T = 325
N = 39
IN_F = 4
OUT_F = 64
NC = 20
E_S = N * 3
E_T = 2 * (T - 1)

BC = 2
TILE_S = 25
TILE_T = 3

_PH = jax.lax.Precision.HIGHEST


def _spatial_parts(se_ref):
    src = se_ref[0:1, :].astype(jnp.int32)
    dst = se_ref[1:2, :].astype(jnp.int32)
    iota = jax.lax.broadcasted_iota(jnp.int32, (N, E_S), 0)
    d1 = (dst == iota).astype(jnp.float32)
    s1 = (src == iota).astype(jnp.float32)
    deg = jnp.sum(d1, axis=1, keepdims=True)
    dis = jnp.where(deg > 0, 1.0 / jnp.sqrt(jnp.maximum(deg, 1e-12)), 0.0)
    dis_src = jnp.sum(s1 * dis, axis=0, keepdims=True)
    nrm = dis_src * jnp.sum(d1 * dis, axis=0, keepdims=True)
    s1_3 = s1.reshape(N, N, 3)
    nrm_3 = nrm.reshape(1, N, 3)
    return s1_3, nrm_3


def _spatial_hop(x, s1_3, nrm_3, m_s):
    acc = None
    for j in range(3):
        g = jnp.dot(x, s1_3[:, :, j], preferred_element_type=jnp.float32,
                    precision=_PH)
        m_s[...] = g * nrm_3[:, :, j]
        mj = m_s[...]
        acc = mj if acc is None else acc + mj
    return acc


def _temporal_parts(te_ref):
    dst = te_ref[1:2, :].astype(jnp.int32)
    iota = jax.lax.broadcasted_iota(jnp.int32, (T, E_T), 0)
    d1 = (dst == iota).astype(jnp.float32)
    deg = jnp.sum(d1, axis=1, keepdims=True)
    dis = jnp.where(deg > 0, 1.0 / jnp.sqrt(jnp.maximum(deg, 1e-12)), 0.0)
    disv = dis.reshape(1, T)
    dis_m1 = jnp.concatenate([disv[:, 1:2], disv[:, :-1]], axis=1)
    dis_p1 = jnp.concatenate([disv[:, 1:], disv[:, -1:]], axis=1)
    io = jax.lax.broadcasted_iota(jnp.int32, (1, T), 1)
    nrm0 = dis_m1 * disv
    nrm1 = jnp.where((io > 0) & (io < T - 1), dis_p1 * disv, 0.0)
    return nrm0, nrm1


def _temporal_hop(x, nrm0, nrm1, m_s):
    g0 = jnp.concatenate([x[:, 1:2], x[:, :-1]], axis=1)
    g1 = jnp.concatenate([x[:, 1:], x[:, -1:]], axis=1)
    m_s[...] = g0 * nrm0
    acc = m_s[...]
    m_s[...] = g1 * nrm1
    return acc + m_s[...]


def _lif_fc(hops, w, br, wfc, cv_s, sp_s, xf_s, p_ref, n_steps, n_nodes, tile):
    rows_per_step = BC * n_nodes
    mem = jnp.zeros((rows_per_step, OUT_F), jnp.float32)
    spike = jnp.zeros((rows_per_step, OUT_F), jnp.float32)
    rtile = tile * IN_F * BC
    for t0 in range(n_steps // tile):
        parts = []
        for hk in hops:
            hk_t = hk[t0 * rtile:(t0 + 1) * rtile]
            hk_t = hk_t.reshape(tile, IN_F, BC, n_nodes)
            parts.append(jnp.transpose(hk_t, (0, 2, 3, 1)))
        cat = jnp.concatenate(parts, axis=3)
        cat = cat.reshape(tile * BC * n_nodes, (K + 1) * IN_F)
        conv = jnp.dot(cat, w, preferred_element_type=jnp.float32)
        conv = conv + br
        cv_s[...] = conv.reshape(tile, rows_per_step, OUT_F)

        def body(i, carry):
            mem, spike = carry
            mem = mem * DECAY * (1.0 - spike) + cv_s[i]
            spike = (mem > THRESH).astype(jnp.float32)
            sp_s[i] = spike.reshape(BC, n_nodes, OUT_F)
            return (mem, spike)
        mem, spike = jax.lax.fori_loop(0, tile, body, (mem, spike))

        xf_s[...] = sp_s[...].reshape(tile * BC, n_nodes, OUT_F)
        xf = xf_s[...].reshape(tile * BC, n_nodes * OUT_F)
        p = jnp.dot(xf, wfc, preferred_element_type=jnp.float32)
        p_ref[:, t0 * tile:(t0 + 1) * tile, :] = jnp.transpose(
            p.reshape(tile, BC, NC), (1, 0, 2))


def _main_kernel(xsp_ref, xtp_ref, w1_ref, b1_ref, w2_ref, b2_ref,
                 wfc_ref, wtfc_ref, se_ref, te_ref, ps_ref, pt_ref,
                 ms_s, mt_s, cvs_s, sps_s, xfs_s, cvt_s, spt_s, xft_s):
    s1_3, nrm_3 = _spatial_parts(se_ref)
    x = jnp.transpose(xsp_ref[...], (1, 0, 2))
    x2 = x.reshape(T * IN_F * BC, N)
    hops = [x2]
    h = x2
    for _ in range(K):
        h = _spatial_hop(h, s1_3, nrm_3, ms_s)
        hops.append(h)
    _lif_fc(hops, w1_ref[...], b1_ref[...], wfc_ref[...],
            cvs_s, sps_s, xfs_s, ps_ref, T, N, TILE_S)

    nrm0, nrm1 = _temporal_parts(te_ref)
    xt = jnp.transpose(xtp_ref[...], (1, 0, 2))
    xt2 = xt.reshape(N * IN_F * BC, T)
    hops_t = [xt2]
    ht = xt2
    for _ in range(K):
        ht = _temporal_hop(ht, nrm0, nrm1, mt_s)
        hops_t.append(ht)
    _lif_fc(hops_t, w2_ref[...], b2_ref[...], wtfc_ref[...],
            cvt_s, spt_s, xft_s, pt_ref, N, T, TILE_T)


def _rec_kernel(ps_ref, pt_ref, bfc_ref, btfc_ref, out_ref):
    def step(p_ref, b):
        def body(i, carry):
            mem, spike, acc = carry
            mem = mem * DECAY * (1.0 - spike) + p_ref[i] + b
            spike = (mem > THRESH).astype(jnp.float32)
            return (mem, spike, acc + spike)
        return body
    z = jnp.zeros((B, NC), jnp.float32)
    _, _, hsum = jax.lax.fori_loop(0, T, step(ps_ref, bfc_ref[...]),
                                   (z, z, z))
    _, _, tsum = jax.lax.fori_loop(0, N, step(pt_ref, btfc_ref[...]),
                                   (z, z, z))
    out_ref[...] = (hsum / float(T) + tsum / float(N)) / 2.0


def kernel(input, W1, b1, W2, b2, Wfc1, bfc1, Wtfc1, btfc1,
           spatial_edge_index, temporal_edge_index):
    data = input
    x4 = jnp.concatenate([data[:, :N], data[:, N:]], axis=2)
    xsp = jnp.transpose(x4, (0, 3, 2, 1)).reshape(B, T * IN_F, N)
    xtp = x4.reshape(B, N * IN_F, T)
    b1r = b1.reshape(1, OUT_F)
    b2r = b2.reshape(1, OUT_F)
    bfc = bfc1.reshape(1, NC)
    btfc = btfc1.reshape(1, NC)

    nchunks = B // BC

    ps, pt = pl.pallas_call(
        _main_kernel,
        grid=(nchunks,),
        in_specs=[
            pl.BlockSpec((BC, T * IN_F, N), lambda c: (c, 0, 0)),
            pl.BlockSpec((BC, N * IN_F, T), lambda c: (c, 0, 0)),
            pl.BlockSpec(((K + 1) * IN_F, OUT_F), lambda c: (0, 0)),
            pl.BlockSpec((1, OUT_F), lambda c: (0, 0)),
            pl.BlockSpec(((K + 1) * IN_F, OUT_F), lambda c: (0, 0)),
            pl.BlockSpec((1, OUT_F), lambda c: (0, 0)),
            pl.BlockSpec((N * OUT_F, NC), lambda c: (0, 0)),
            pl.BlockSpec((T * OUT_F, NC), lambda c: (0, 0)),
            pl.BlockSpec((2, E_S), lambda c: (0, 0)),
            pl.BlockSpec((2, E_T), lambda c: (0, 0)),
        ],
        out_specs=[
            pl.BlockSpec((BC, T, NC), lambda c: (c, 0, 0)),
            pl.BlockSpec((BC, N, NC), lambda c: (c, 0, 0)),
        ],
        out_shape=[
            jax.ShapeDtypeStruct((B, T, NC), jnp.float32),
            jax.ShapeDtypeStruct((B, N, NC), jnp.float32),
        ],
        scratch_shapes=[
            pltpu.VMEM((T * BC * IN_F, N), jnp.float32),
            pltpu.VMEM((N * BC * IN_F, T), jnp.float32),
            pltpu.VMEM((TILE_S, BC * N, OUT_F), jnp.float32),
            pltpu.VMEM((TILE_S, BC, N, OUT_F), jnp.float32),
            pltpu.VMEM((TILE_S * BC, N, OUT_F), jnp.float32),
            pltpu.VMEM((TILE_T, BC * T, OUT_F), jnp.float32),
            pltpu.VMEM((TILE_T, BC, T, OUT_F), jnp.float32),
            pltpu.VMEM((TILE_T * BC, T, OUT_F), jnp.float32),
        ],
        compiler_params=pltpu.CompilerParams(
            dimension_semantics=("arbitrary",),
        ),
    )(xsp, xtp, W1, b1r, W2, b2r, Wfc1, Wtfc1,
      spatial_edge_index, temporal_edge_index)

    pst = jnp.transpose(ps, (1, 0, 2))
    ptt = jnp.transpose(pt, (1, 0, 2))
    out = pl.pallas_call(
        _rec_kernel,
        in_specs=[
            pl.BlockSpec((T, B, NC), lambda: (0, 0, 0)),
            pl.BlockSpec((N, B, NC), lambda: (0, 0, 0)),
            pl.BlockSpec((1, NC), lambda: (0, 0)),
            pl.BlockSpec((1, NC), lambda: (0, 0)),
        ],
        out_specs=pl.BlockSpec((B, NC), lambda: (0, 0)),
        out_shape=jax.ShapeDtypeStruct((B, NC), jnp.float32),
    )(pst, ptt, bfc, btfc)
    return out
